# Initial kernel scaffold; baseline (speedup 1.0000x reference)
import jax
import jax.numpy as jnp
from jax.experimental import pallas as pl


def kernel(x, edge_index, W_start, b_start, W_cat, b_cat, W1, attn_l1, attn_r1, bias1, W2, attn_l2, attn_r2, bias2):
    raise NotImplementedError("write your pallas kernel here")


# speedup vs baseline: 1.0229x; 1.0229x over previous
import functools
import numpy as np
import jax
import jax.numpy as jnp
from jax import lax
from jax.experimental import pallas as pl
from jax.experimental.pallas import tpu as pltpu

NEG_SLOPE_EDGE = 0.2
NEG_SLOPE_PRE = 0.01


def _leaky(x, slope):
    return jnp.maximum(x, slope * x)


def _elu(x):
    return jnp.where(x > 0, x, jnp.exp(jnp.minimum(x, 0.0)) - 1.0)


def _epilogue_block(acc4b, biasb):
    total = None
    for h in range(8):
        g, j = h // 2, h % 2
        s = jnp.maximum(acc4b[g, :, 192 + j:193 + j], 1e-38)
        v = acc4b[g, :, 96 * j:96 * j + 96] / s + biasb[h:h + 1, :]
        e = _elu(v)
        total = e if total is None else total + e
    return total * 0.125


def _kernel_a(xr_ref, wb1_ref, b1_ref, wb2_ref, b2_ref, w1_ref, alr_ref,
              h_ref, feat_ref, elr_ref, m_ref, macc):
    i = pl.program_id(0)
    xb = xr_ref[...]
    z1 = jnp.dot(xb, wb1_ref[...], preferred_element_type=jnp.float32) + b1_ref[...]
    z2 = jnp.dot(xb, wb2_ref[...], preferred_element_type=jnp.float32) + b2_ref[...]
    hb = z1 + _leaky(z2, NEG_SLOPE_PRE)
    h_ref[...] = hb
    fb = jnp.dot(hb, w1_ref[...], preferred_element_type=jnp.float32)
    feat_ref[...] = fb
    elrb = jnp.dot(fb, alr_ref[...], preferred_element_type=jnp.float32)
    elr_ref[...] = elrb
    mel = jnp.max(elrb[:, :8], axis=0, keepdims=True)
    m2 = jnp.concatenate([mel, mel], axis=1)

    @pl.when(i == 0)
    def _():
        macc[...] = jnp.full((1, 16), -jnp.inf, jnp.float32)

    macc[...] = jnp.maximum(macc[...], m2)
    m_ref[...] = macc[...]


def _kernel_b(acc_ref, bias_ref, w2_ref, alr_ref,
              feat_ref, elr_ref, m_ref, macc):
    i = pl.program_id(0)
    h1b = _epilogue_block(acc_ref[...], bias_ref[...])
    fb = jnp.dot(h1b, w2_ref[...], preferred_element_type=jnp.float32)
    feat_ref[...] = fb
    elrb = jnp.dot(fb, alr_ref[...], preferred_element_type=jnp.float32)
    elr_ref[...] = elrb
    mel = jnp.max(elrb[:, :8], axis=0, keepdims=True)
    m2 = jnp.concatenate([mel, mel], axis=1)

    @pl.when(i == 0)
    def _():
        macc[...] = jnp.full((1, 16), -jnp.inf, jnp.float32)

    macc[...] = jnp.maximum(macc[...], m2)
    m_ref[...] = macc[...]


def _kernel_c(acc_ref, bias_ref, h_ref, out_ref):
    h2b = _epilogue_block(acc_ref[...], bias_ref[...])
    out_ref[...] = h_ref[...] + h2b


def _run_a(xr, Wb1, b1r, Wb2, b2r, W1, ALR, N, Nb):
    nblk = N // Nb
    full = lambda shape: pl.BlockSpec(shape, lambda i: (0,) * len(shape))
    return pl.pallas_call(
        _kernel_a,
        grid=(nblk,),
        in_specs=[
            pl.BlockSpec((Nb, 24), lambda i: (i, 0)),
            full((24, 96)), full((1, 96)), full((24, 96)), full((1, 96)),
            full((96, 768)), full((768, 16)),
        ],
        out_specs=[
            pl.BlockSpec((Nb, 96), lambda i: (i, 0)),
            pl.BlockSpec((Nb, 768), lambda i: (i, 0)),
            pl.BlockSpec((Nb, 16), lambda i: (i, 0)),
            pl.BlockSpec((1, 16), lambda i: (0, 0)),
        ],
        out_shape=[
            jax.ShapeDtypeStruct((N, 96), jnp.float32),
            jax.ShapeDtypeStruct((N, 768), jnp.float32),
            jax.ShapeDtypeStruct((N, 16), jnp.float32),
            jax.ShapeDtypeStruct((1, 16), jnp.float32),
        ],
        scratch_shapes=[pltpu.VMEM((1, 16), jnp.float32)],
    )(xr, Wb1, b1r, Wb2, b2r, W1, ALR)


def _run_b(acc4, bias, W2, ALR2, N, Nb):
    nblk = N // Nb
    full = lambda shape: pl.BlockSpec(shape, lambda i: (0,) * len(shape))
    return pl.pallas_call(
        _kernel_b,
        grid=(nblk,),
        in_specs=[
            pl.BlockSpec((4, Nb, 200), lambda i: (0, i, 0)),
            full((8, 96)), full((96, 768)), full((768, 16)),
        ],
        out_specs=[
            pl.BlockSpec((Nb, 768), lambda i: (i, 0)),
            pl.BlockSpec((Nb, 16), lambda i: (i, 0)),
            pl.BlockSpec((1, 16), lambda i: (0, 0)),
        ],
        out_shape=[
            jax.ShapeDtypeStruct((N, 768), jnp.float32),
            jax.ShapeDtypeStruct((N, 16), jnp.float32),
            jax.ShapeDtypeStruct((1, 16), jnp.float32),
        ],
        scratch_shapes=[pltpu.VMEM((1, 16), jnp.float32)],
    )(acc4, bias, W2, ALR2)


def _run_c(acc4, bias, h, N, Nb):
    nblk = N // Nb
    full = lambda shape: pl.BlockSpec(shape, lambda i: (0,) * len(shape))
    return pl.pallas_call(
        _kernel_c,
        grid=(nblk,),
        in_specs=[
            pl.BlockSpec((4, Nb, 200), lambda i: (0, i, 0)),
            full((8, 96)),
            pl.BlockSpec((Nb, 96), lambda i: (i, 0)),
        ],
        out_specs=pl.BlockSpec((Nb, 96), lambda i: (i, 0)),
        out_shape=jax.ShapeDtypeStruct((N, 96), jnp.float32),
    )(acc4, bias, h)


def _graph_stage_jnp(feat, elr, M16, src, dst, N):
    el = elr[:, :8]
    er = elr[:, 8:]
    M = M16[0, :8]
    t = el[src] + er[dst]
    e = _leaky(t, NEG_SLOPE_EDGE)
    q = M[None, :] + er[dst]
    c = _leaky(q, NEG_SLOPE_EDGE)
    w = jnp.exp(e - c)
    featH = feat.reshape(N, 8, 96)
    acc8 = jax.ops.segment_sum(w[:, :, None] * featH[src], dst, num_segments=N)
    s8 = jax.ops.segment_sum(w, dst, num_segments=N)
    blocks = []
    for g in range(4):
        blk = jnp.concatenate(
            [acc8[:, 2 * g:2 * g + 2, :].reshape(N, 192),
             s8[:, 2 * g:2 * g + 2], jnp.zeros((N, 6), jnp.float32)], axis=1)
        blocks.append(blk)
    return jnp.stack(blocks, axis=0)


def kernel(x, edge_index, W_start, b_start, W_cat, b_cat,
           W1, attn_l1, attn_r1, bias1, W2, attn_l2, attn_r2, bias2):
    B, C, N, S = x.shape
    emb = W_start.shape[0]
    H, D = attn_l1.shape
    Nb = 400

    xr = jnp.transpose(x[0], (1, 0, 2)).reshape(N, C * S)
    eyeS = jnp.eye(S, dtype=jnp.float32)
    Wb1 = (W_start.T[:, None, :, None] * eyeS[None, :, None, :]).reshape(C * S, emb * S)
    Wb2 = (W_cat.T[:, None, :, None] * eyeS[None, :, None, :]).reshape(C * S, emb * S)
    b1r = jnp.broadcast_to(b_start[:, None], (emb, S)).reshape(1, emb * S)
    b2r = jnp.broadcast_to(b_cat[:, None], (emb, S)).reshape(1, emb * S)

    eyeH = jnp.eye(H, dtype=jnp.float32)

    def attn_mat(al, ar):
        zl = al[:, :, None] * eyeH[:, None, :]
        zr = ar[:, :, None] * eyeH[:, None, :]
        return jnp.concatenate([zl, zr], axis=2).reshape(H * D, 2 * H)

    ALR1 = attn_mat(attn_l1, attn_r1)
    ALR2 = attn_mat(attn_l2, attn_r2)
    src = edge_index[0].astype(jnp.int32)
    dst = edge_index[1].astype(jnp.int32)

    h, feat1, elr1, M1 = _run_a(xr, Wb1, b1r, Wb2, b2r, W1, ALR1, N, Nb)

    acc1 = _graph_stage_jnp(feat1, elr1, M1, src, dst, N)

    feat2, elr2, M2 = _run_b(acc1, bias1, W2, ALR2, N, Nb)

    acc2 = _graph_stage_jnp(feat2, elr2, M2, src, dst, N)

    out96 = _run_c(acc2, bias2, h, N, Nb)

    out = jnp.transpose(out96.reshape(N, emb, S), (1, 0, 2))[None]
    return out


# speedup vs baseline: 6.6052x; 6.4571x over previous
import functools
import numpy as np
import jax
import jax.numpy as jnp
from jax import lax
from jax.experimental import pallas as pl
from jax.experimental.pallas import tpu as pltpu
from jax.experimental.pallas import tpu_sc as plsc

NEG_SLOPE_EDGE = 0.2
NEG_SLOPE_PRE = 0.01


def _leaky(x, slope):
    return jnp.maximum(x, slope * x)


def _elu(x):
    return jnp.where(x > 0, x, jnp.exp(jnp.minimum(x, 0.0)) - 1.0)


def _epilogue_block(acc8b, biasb):
    total = None
    for h in range(8):
        s = jnp.maximum(acc8b[h, :, 96:97], 1e-38)
        v = acc8b[h, :, 0:96] / s + biasb[h:h + 1, :]
        e = _elu(v)
        total = e if total is None else total + e
    return total * 0.125


def _pad_feat(fb):
    nb = fb.shape[0]
    z = jnp.zeros((nb, 32), jnp.float32)
    parts = []
    for hh in range(8):
        parts.append(fb[:, 96 * hh:96 * hh + 96])
        parts.append(z)
    return jnp.concatenate(parts, axis=1)


def _kernel_a(xr_ref, wb1_ref, b1_ref, wb2_ref, b2_ref, w1_ref, alr_ref,
              h_ref, feat_ref, elr_ref, m_ref, macc):
    i = pl.program_id(0)
    xb = xr_ref[...]
    z1 = jnp.dot(xb, wb1_ref[...], preferred_element_type=jnp.float32) + b1_ref[...]
    z2 = jnp.dot(xb, wb2_ref[...], preferred_element_type=jnp.float32) + b2_ref[...]
    hb = z1 + _leaky(z2, NEG_SLOPE_PRE)
    h_ref[...] = hb
    fb = jnp.dot(hb, w1_ref[...], preferred_element_type=jnp.float32)
    feat_ref[...] = _pad_feat(fb)
    elrb = jnp.dot(fb, alr_ref[...], preferred_element_type=jnp.float32)
    elr_ref[...] = jnp.concatenate(
        [elrb, jnp.zeros((elrb.shape[0], 112), jnp.float32)], axis=1)
    mel = jnp.max(elrb[:, :8], axis=0, keepdims=True)
    m2 = jnp.concatenate([mel, mel], axis=1)

    @pl.when(i == 0)
    def _():
        macc[...] = jnp.full((1, 16), -jnp.inf, jnp.float32)

    macc[...] = jnp.maximum(macc[...], m2)
    m_ref[...] = macc[...]


def _kernel_b(acc_ref, bias_ref, w2_ref, alr_ref,
              feat_ref, elr_ref, m_ref, macc):
    i = pl.program_id(0)
    h1b = _epilogue_block(acc_ref[...], bias_ref[...])
    fb = jnp.dot(h1b, w2_ref[...], preferred_element_type=jnp.float32)
    feat_ref[...] = _pad_feat(fb)
    elrb = jnp.dot(fb, alr_ref[...], preferred_element_type=jnp.float32)
    elr_ref[...] = jnp.concatenate(
        [elrb, jnp.zeros((elrb.shape[0], 112), jnp.float32)], axis=1)
    mel = jnp.max(elrb[:, :8], axis=0, keepdims=True)
    m2 = jnp.concatenate([mel, mel], axis=1)

    @pl.when(i == 0)
    def _():
        macc[...] = jnp.full((1, 16), -jnp.inf, jnp.float32)

    macc[...] = jnp.maximum(macc[...], m2)
    m_ref[...] = macc[...]


def _kernel_c(acc_ref, bias_ref, h_ref, out_ref):
    h2b = _epilogue_block(acc_ref[...], bias_ref[...])
    out_ref[...] = h_ref[...] + h2b


def _run_a(xr, Wb1, b1r, Wb2, b2r, W1, ALR, N, Nb):
    nblk = N // Nb
    full = lambda shape: pl.BlockSpec(shape, lambda i: (0,) * len(shape))
    return pl.pallas_call(
        _kernel_a,
        grid=(nblk,),
        in_specs=[
            pl.BlockSpec((Nb, 24), lambda i: (i, 0)),
            full((24, 96)), full((1, 96)), full((24, 96)), full((1, 96)),
            full((96, 768)), full((768, 16)),
        ],
        out_specs=[
            pl.BlockSpec((Nb, 96), lambda i: (i, 0)),
            pl.BlockSpec((Nb, 1024), lambda i: (i, 0)),
            pl.BlockSpec((Nb, 128), lambda i: (i, 0)),
            pl.BlockSpec((1, 16), lambda i: (0, 0)),
        ],
        out_shape=[
            jax.ShapeDtypeStruct((N, 96), jnp.float32),
            jax.ShapeDtypeStruct((N, 1024), jnp.float32),
            jax.ShapeDtypeStruct((N, 128), jnp.float32),
            jax.ShapeDtypeStruct((1, 16), jnp.float32),
        ],
        scratch_shapes=[pltpu.VMEM((1, 16), jnp.float32)],
    )(xr, Wb1, b1r, Wb2, b2r, W1, ALR)


def _run_b(acc4, bias, W2, ALR2, N, Nb):
    nblk = N // Nb
    full = lambda shape: pl.BlockSpec(shape, lambda i: (0,) * len(shape))
    return pl.pallas_call(
        _kernel_b,
        grid=(nblk,),
        in_specs=[
            pl.BlockSpec((8, Nb, 128), lambda i: (0, i, 0)),
            full((8, 96)), full((96, 768)), full((768, 16)),
        ],
        out_specs=[
            pl.BlockSpec((Nb, 1024), lambda i: (i, 0)),
            pl.BlockSpec((Nb, 128), lambda i: (i, 0)),
            pl.BlockSpec((1, 16), lambda i: (0, 0)),
        ],
        out_shape=[
            jax.ShapeDtypeStruct((N, 1024), jnp.float32),
            jax.ShapeDtypeStruct((N, 128), jnp.float32),
            jax.ShapeDtypeStruct((1, 16), jnp.float32),
        ],
        scratch_shapes=[pltpu.VMEM((1, 16), jnp.float32)],
    )(acc4, bias, W2, ALR2)


def _run_c(acc4, bias, h, N, Nb):
    nblk = N // Nb
    full = lambda shape: pl.BlockSpec(shape, lambda i: (0,) * len(shape))
    return pl.pallas_call(
        _kernel_c,
        grid=(nblk,),
        in_specs=[
            pl.BlockSpec((8, Nb, 128), lambda i: (0, i, 0)),
            full((8, 96)),
            pl.BlockSpec((Nb, 96), lambda i: (i, 0)),
        ],
        out_specs=pl.BlockSpec((Nb, 96), lambda i: (i, 0)),
        out_shape=jax.ShapeDtypeStruct((N, 96), jnp.float32),
    )(acc4, bias, h)


_GATHER_DNUMS = lax.GatherDimensionNumbers(
    offset_dims=(), collapsed_slice_dims=(0,), start_index_map=(0,))


def _vperm(vec, idx):
    return lax.gather(vec, idx[:, None], _GATHER_DNUMS, (1,),
                      mode=lax.GatherScatterMode.PROMISE_IN_BOUNDS)


def _w_body(elr_hbm, src_hbm, dst_hbm, m_hbm, w_hbm,
            m_v, sidx, didx, elbuf, erbuf, wbuf, sem1, sem2):
    E = src_hbm.shape[0]
    ept = E // 32
    nch = ept // 40
    c = lax.axis_index("c")
    s = lax.axis_index("s")
    wid = s * 2 + c
    pltpu.sync_copy(m_hbm, m_v)
    mvec = m_v[0]
    i16 = lax.iota(jnp.int32, 16)
    idx_lo = i16 & 7
    idx_hi = idx_lo + 8
    mask8 = i16 < 8

    def chunk(ch, _):
        base = wid * ept + ch * 40
        pltpu.sync_copy(src_hbm.at[pl.ds(base, 40)], sidx)
        pltpu.sync_copy(dst_hbm.at[pl.ds(base, 40)], didx)
        pltpu.async_copy(elr_hbm.at[sidx], elbuf, sem1).wait()
        pltpu.async_copy(elr_hbm.at[didx], erbuf, sem2).wait()

        def epair(i, _):
            u_a = elbuf[2 * i, 0:16]
            u_b = elbuf[2 * i + 1, 0:16]
            v_a = erbuf[2 * i, 0:16]
            v_b = erbuf[2 * i + 1, 0:16]
            u2 = jnp.where(mask8, u_a, _vperm(u_b, idx_lo))
            v2 = jnp.where(mask8, _vperm(v_a, idx_hi), v_b)
            t = u2 + v2
            e16 = jnp.maximum(t, NEG_SLOPE_EDGE * t)
            q = mvec + v2
            c16 = jnp.maximum(q, NEG_SLOPE_EDGE * q)
            w16 = jnp.exp(e16 - c16)
            wbuf[pl.ds(i * 16, 16)] = w16
            return 0

        lax.fori_loop(0, 20, epair, 0, unroll=2)
        pltpu.sync_copy(wbuf.at[pl.ds(0, 320)],
                        w_hbm.at[pl.ds(base * 8, 320)])
        return 0

    lax.fori_loop(0, nch, chunk, 0)


def _run_w(elr, src, dst, M16, E):
    mesh = plsc.VectorSubcoreMesh(core_axis_name="c", subcore_axis_name="s")
    f = pl.kernel(
        _w_body,
        out_type=jax.ShapeDtypeStruct((E * 8,), jnp.float32),
        mesh=mesh,
        scratch_types=[
            pltpu.VMEM((1, 16), jnp.float32),
            pltpu.VMEM((40,), jnp.int32),
            pltpu.VMEM((40,), jnp.int32),
            pltpu.VMEM((40, 128), jnp.float32),
            pltpu.VMEM((40, 128), jnp.float32),
            pltpu.VMEM((336,), jnp.float32),
            pltpu.SemaphoreType.DMA,
            pltpu.SemaphoreType.DMA,
        ],
    )
    return f(elr, src, dst, M16)


def _big_body(feat8_hbm, src_hbm, dst_hbm, w_hbm, acc_out,
              sidx, fidx, didx2, wv, featb, scaled, acc, semg):
    E = src_hbm.shape[0]
    ept = E // 16
    nch = ept // 80
    N = acc_out.shape[1]
    rpt = 624
    c = lax.axis_index("c")
    s = lax.axis_index("s")
    i16 = lax.iota(jnp.int32, 16)
    zero16 = jnp.zeros((16,), jnp.float32)

    if True:
        for p in range(4):
            hd = 4 * c + p
            bidx = i16 * 0 + hd
            def zrow(i, _):
                for j in range(8):
                    scaled[i, 16 * j:16 * j + 16] = zero16
                return 0
            lax.fori_loop(0, 80, zrow, 0)
            for k in range(7):
                pltpu.sync_copy(scaled, acc.at[pl.ds(s * rpt + k * 80, 80)])
            pltpu.sync_copy(scaled.at[pl.ds(0, 64)],
                            acc.at[pl.ds(s * rpt + 560, 64)])

            @pl.when(s == 15)
            def _():
                pltpu.sync_copy(scaled.at[pl.ds(0, 16)],
                                acc.at[pl.ds(16 * rpt, 16)])
            plsc.subcore_barrier()

            def chunk(ch, _):
                base = s * ept + ch * 80
                pltpu.sync_copy(src_hbm.at[pl.ds(base, 80)], sidx)
                pltpu.sync_copy(dst_hbm.at[pl.ds(base, 80)], didx2.at[0])
                pltpu.sync_copy(w_hbm.at[pl.ds(base * 8, 640)],
                                wv.at[pl.ds(0, 640)])

                def mkidx(k, _):
                    fidx[pl.ds(16 * k, 16)] = sidx[pl.ds(16 * k, 16)] * 8 + hd
                    return 0
                lax.fori_loop(0, 5, mkidx, 0)
                pltpu.async_copy(feat8_hbm.at[fidx], featb, semg).wait()

                def edge(e, _):
                    wwin = wv[pl.ds(e * 8, 16)]
                    b0 = _vperm(wwin, bidx)
                    for j in range(6):
                        f = featb[e, 16 * j:16 * j + 16]
                        scaled[e, 16 * j:16 * j + 16] = f * b0
                    wps = jnp.where(i16 == 0, b0, zero16)
                    scaled[e, 96:112] = wps
                    return 0

                lax.fori_loop(0, 80, edge, 0, unroll=2)
                pltpu.sync_copy(scaled, acc.at[didx2.at[0]], add=True)
                return 0

            lax.fori_loop(0, nch, chunk, 0)
            plsc.subcore_barrier()

            for k in range(7):
                pltpu.sync_copy(acc.at[pl.ds(s * rpt + k * 80, 80)],
                                acc_out.at[hd, pl.ds(s * rpt + k * 80, 80)])
            pltpu.sync_copy(acc.at[pl.ds(s * rpt + 560, 64)],
                            acc_out.at[hd, pl.ds(s * rpt + 560, 64)])

            @pl.when(s == 15)
            def _():
                pltpu.sync_copy(acc.at[pl.ds(16 * rpt, 16)],
                                acc_out.at[hd, pl.ds(16 * rpt, 16)])
            plsc.subcore_barrier()


def _run_big(feat8, src, dst, w8, N):
    mesh = plsc.VectorSubcoreMesh(core_axis_name="c", subcore_axis_name="s")
    f = pl.kernel(
        _big_body,
        out_type=jax.ShapeDtypeStruct((8, N, 128), jnp.float32),
        mesh=mesh,
        scratch_types=[
            pltpu.VMEM((80,), jnp.int32),
            pltpu.VMEM((80,), jnp.int32),
            pltpu.VMEM((1, 80), jnp.int32),
            pltpu.VMEM((656,), jnp.float32),
            pltpu.VMEM((80, 128), jnp.float32),
            pltpu.VMEM((80, 128), jnp.float32),
            pltpu.VMEM_SHARED((10000, 128), jnp.float32),
            pltpu.SemaphoreType.DMA,
        ],
    )
    return f(feat8, src, dst, w8)


def _graph_stage_sc(feat, elr, M16, src, dst, N, E):
    w8 = _run_w(elr, src, dst, M16, E)
    feat8 = feat.reshape(N * 8, 128)
    return _run_big(feat8, src, dst, w8, N)


def kernel(x, edge_index, W_start, b_start, W_cat, b_cat,
           W1, attn_l1, attn_r1, bias1, W2, attn_l2, attn_r2, bias2):
    B, C, N, S = x.shape
    emb = W_start.shape[0]
    H, D = attn_l1.shape
    Nb = 400

    xr = jnp.transpose(x[0], (1, 0, 2)).reshape(N, C * S)
    eyeS = jnp.eye(S, dtype=jnp.float32)
    Wb1 = (W_start.T[:, None, :, None] * eyeS[None, :, None, :]).reshape(C * S, emb * S)
    Wb2 = (W_cat.T[:, None, :, None] * eyeS[None, :, None, :]).reshape(C * S, emb * S)
    b1r = jnp.broadcast_to(b_start[:, None], (emb, S)).reshape(1, emb * S)
    b2r = jnp.broadcast_to(b_cat[:, None], (emb, S)).reshape(1, emb * S)

    eyeH = jnp.eye(H, dtype=jnp.float32)

    def attn_mat(al, ar):
        zl = al[:, :, None] * eyeH[:, None, :]
        zr = ar[:, :, None] * eyeH[:, None, :]
        return jnp.concatenate([zl, zr], axis=2).reshape(H * D, 2 * H)

    ALR1 = attn_mat(attn_l1, attn_r1)
    ALR2 = attn_mat(attn_l2, attn_r2)
    src = edge_index[0].astype(jnp.int32)
    dst = edge_index[1].astype(jnp.int32)

    h, feat1, elr1, M1 = _run_a(xr, Wb1, b1r, Wb2, b2r, W1, ALR1, N, Nb)

    E = src.shape[0]
    acc1 = _graph_stage_sc(feat1, elr1, M1, src, dst, N, E)

    feat2, elr2, M2 = _run_b(acc1, bias1, W2, ALR2, N, Nb)

    acc2 = _graph_stage_sc(feat2, elr2, M2, src, dst, N, E)

    out96 = _run_c(acc2, bias2, h, N, Nb)

    out = jnp.transpose(out96.reshape(N, emb, S), (1, 0, 2))[None]
    return out
